# Initial kernel scaffold; baseline (speedup 1.0000x reference)
import jax
import jax.numpy as jnp
from jax import lax
from jax.experimental import pallas as pl
from jax.experimental.pallas import tpu as pltpu
from jax.experimental.pallas import tpu_sc as plsc

VOCAB = 1028
NUM_CH = 9
HID = 2048
NUM_WORKERS = 32
TOK_BLK = 16


def _sc_body(idx_hbm, table_hbm, out_hbm, idx_v, acc_v, gsem):
    wid = lax.axis_index("s") * 2 + lax.axis_index("c")
    nblk = idx_hbm.shape[1] // NUM_WORKERS

    pltpu.sync_copy(idx_hbm.at[:, pl.ds(wid * nblk, nblk), :], idx_v)

    for c in range(1, NUM_CH):
        def _add(b, _, c=c):
            idx_v[c, b] = idx_v[c, b] + c * VOCAB
            return 0
        lax.fori_loop(0, nblk, _add, 0)

    def _blk(b, _):
        pltpu.async_copy(table_hbm.at[idx_v.at[0, b]], acc_v, gsem).wait()
        for c in range(1, NUM_CH):
            pltpu.async_copy(table_hbm.at[idx_v.at[c, b]], acc_v, gsem, add=True)
        for c in range(1, NUM_CH):
            pltpu.make_async_copy(table_hbm.at[idx_v.at[c, b]], acc_v, gsem).wait()
        base = wid * (nblk * TOK_BLK) + b * TOK_BLK
        pltpu.sync_copy(acc_v, out_hbm.at[pl.ds(base, TOK_BLK), :])
        return 0

    lax.fori_loop(0, nblk, _blk, 0)


def kernel(audio_codes, embed_weight):
    b, s, c = audio_codes.shape
    tokens = audio_codes.reshape(b * s, c).T.reshape(c, (b * s) // TOK_BLK, TOK_BLK)
    mesh = plsc.VectorSubcoreMesh(core_axis_name="c", subcore_axis_name="s")
    nblk = (b * s) // TOK_BLK // NUM_WORKERS
    out = pl.kernel(
        _sc_body,
        out_type=jax.ShapeDtypeStruct((b * s, HID), jnp.float32),
        mesh=mesh,
        scratch_types=[
            pltpu.VMEM((NUM_CH, nblk, TOK_BLK), jnp.int32),
            pltpu.VMEM((TOK_BLK, HID), jnp.float32),
            pltpu.SemaphoreType.DMA,
        ],
    )(tokens, embed_weight)
    return out.reshape(b, s, HID)


# speedup vs baseline: 2.8857x; 2.8857x over previous
import jax
import jax.numpy as jnp
from jax import lax
from jax.experimental import pallas as pl
from jax.experimental.pallas import tpu as pltpu
from jax.experimental.pallas import tpu_sc as plsc

VOCAB = 1028
NUM_CH = 9
HID = 2048
NUM_WORKERS = 32
GRP = 2
IDX_PAD = 24
ROWS = NUM_CH * GRP


def _sc_body(idx_hbm, table_hbm, out_hbm, idx_v, stage_v, obuf_v,
             gsem0, gsem1, osem0, osem1):
    sid = lax.axis_index("s")
    cid = lax.axis_index("c")
    wid = sid * 2 + cid
    toks = out_hbm.shape[0] // NUM_WORKERS
    ng = toks // GRP

    pltpu.sync_copy(idx_hbm.at[wid], idx_v)

    gsems = (gsem0, gsem1)
    osems = (osem0, osem1)

    def _gather(g, buf):
        pltpu.async_copy(
            table_hbm.at[idx_v.at[pl.ds(g * IDX_PAD, ROWS)]],
            stage_v.at[buf], gsems[buf])

    def _gather_wait(g, buf):
        pltpu.make_async_copy(
            table_hbm.at[idx_v.at[pl.ds(g * IDX_PAD, ROWS)]],
            stage_v.at[buf], gsems[buf]).wait()

    def _out_start(g, buf):
        pltpu.async_copy(
            obuf_v.at[buf],
            out_hbm.at[pl.ds(wid * toks + g * GRP, GRP), :], osems[buf])

    def _out_wait(g, buf):
        pltpu.make_async_copy(
            obuf_v.at[buf],
            out_hbm.at[pl.ds(wid * toks + g * GRP, GRP), :], osems[buf]).wait()

    _gather(0, 0)

    def _grp2(g2, _):
        for b in range(2):
            g = g2 * 2 + b
            nb = 1 - b

            @pl.when(g + 1 < ng)
            def _():
                _gather(g + 1, nb)

            _gather_wait(g, b)

            @pl.when(g >= 2)
            def _():
                _out_wait(g, b)

            def _ck(k, _, b=b):
                sl = pl.ds(k * 16, 16)
                for t in range(GRP):
                    acc = stage_v[b, t, sl]
                    for c in range(1, NUM_CH):
                        acc = acc + stage_v[b, GRP * c + t, sl]
                    obuf_v[b, t, sl] = acc
                return 0
            lax.fori_loop(0, HID // 16, _ck, 0)

            _out_start(g, b)
        return 0

    lax.fori_loop(0, ng // 2, _grp2, 0)
    _out_wait(ng - 2, 0)
    _out_wait(ng - 1, 1)


def kernel(audio_codes, embed_weight):
    b, s, c = audio_codes.shape
    bs = b * s
    toks = bs // NUM_WORKERS
    ng = toks // GRP
    grp = audio_codes.reshape(NUM_WORKERS, ng, GRP, c).transpose(0, 1, 3, 2)
    grp = grp.reshape(NUM_WORKERS, ng, ROWS)
    offs = jnp.arange(NUM_CH, dtype=jnp.int32) * VOCAB
    grp = grp + jnp.repeat(offs, GRP)[None, None, :]
    grp = jnp.pad(grp, ((0, 0), (0, 0), (0, IDX_PAD - ROWS)))
    idx_arr = grp.reshape(NUM_WORKERS, ng * IDX_PAD)

    mesh = plsc.VectorSubcoreMesh(core_axis_name="c", subcore_axis_name="s")
    out = pl.kernel(
        _sc_body,
        out_type=jax.ShapeDtypeStruct((bs, HID), jnp.float32),
        mesh=mesh,
        scratch_types=[
            pltpu.VMEM((ng * IDX_PAD,), jnp.int32),
            pltpu.VMEM((2, ROWS, HID), jnp.float32),
            pltpu.VMEM((2, GRP, HID), jnp.float32),
            pltpu.SemaphoreType.DMA,
            pltpu.SemaphoreType.DMA,
            pltpu.SemaphoreType.DMA,
            pltpu.SemaphoreType.DMA,
        ],
    )(idx_arr, embed_weight)
    return out.reshape(b, s, HID)
